# baseline (speedup 1.0000x reference)
import functools

import jax
import jax.numpy as jnp
from jax.experimental import pallas as pl
from jax.experimental.pallas import tpu as pltpu

_EPS = 1e-8


def _simrel_tile(ca_t_ref, x_ref, lab_ref, out_ref):
    ca_t = ca_t_ref[...]
    ca_sq = jnp.sum(ca_t * ca_t, axis=0, keepdims=True)
    ca_norm = jnp.sqrt(ca_sq)
    ca_unit = ca_t / jnp.maximum(ca_norm, _EPS)
    has_inf = jnp.any(jnp.isinf(ca_t), axis=0, keepdims=True)

    x = x_ref[...]
    raw = jnp.dot(x, ca_unit, preferred_element_type=jnp.float32)
    x_norm = jnp.sqrt(jnp.sum(x * x, axis=1, keepdims=True))
    cos = raw / jnp.maximum(x_norm, _EPS)

    labels = lab_ref[0]
    mt = cos.shape[0]
    k = cos.shape[1]
    kidx = jax.lax.broadcasted_iota(jnp.int32, (mt, k), 1)
    match = labels.reshape(mt, 1) == kidx
    uninit = jnp.where(match, jnp.float32(1.0), jnp.float32(-1.0))
    out_ref[...] = jnp.where(has_inf, uninit, cos)


@functools.partial(jax.jit, static_argnames=())
def kernel(inputs, labels, class_avgs):
    b, t, d = inputs.shape
    k = class_avgs.shape[0]
    m = b * t
    mt = 1024
    n_tiles = m // mt

    x2 = inputs.reshape(m, d)
    lab3 = labels.astype(jnp.int32).reshape(n_tiles, 1, mt)
    ca_t = class_avgs.T

    out = pl.pallas_call(
        _simrel_tile,
        grid=(n_tiles,),
        in_specs=[
            pl.BlockSpec((d, k), lambda i: (0, 0)),
            pl.BlockSpec((mt, d), lambda i: (i, 0)),
            pl.BlockSpec((1, 1, mt), lambda i: (i, 0, 0)),
        ],
        out_specs=pl.BlockSpec((mt, k), lambda i: (i, 0)),
        out_shape=jax.ShapeDtypeStruct((m, k), jnp.float32),
        compiler_params=pltpu.CompilerParams(
            dimension_semantics=("parallel",),
        ),
    )(ca_t, x2, lab3)
    return out.reshape(b, t, k)
